# baseline (speedup 1.0000x reference)
import functools

import numpy as np
import jax
import jax.numpy as jnp
from jax import lax
from jax.experimental import pallas as pl
from jax.experimental.pallas import tpu as pltpu
from jax.experimental.pallas import tpu_sc as plsc

F = 26
D = 16
B = 16384
H1 = 256
H2 = 256
EMB = F * D
NPAIR = F * (F - 1) // 2
DNN_IN = EMB + NPAIR

NC = 2
NS = 16
NW = NC * NS
TOT = B * F
SUB = 128
CHUNK = 13 * SUB
NSUB = CHUNK // SUB


DT_CB = 125
DT_GRID = 8125 // DT_CB


def _detile_body(g_ref, out_ref):
    x = g_ref[...]
    S = DT_CB * 16
    x8 = jnp.concatenate([x[:, p * S:(p + 1) * S] for p in range(8)], axis=0)
    out_ref[...] = x8.T


def _detile(emT):
    return pl.pallas_call(
        _detile_body,
        grid=(DT_GRID,),
        in_specs=[pl.BlockSpec((16, DT_CB * 128), lambda i: (0, i))],
        out_specs=pl.BlockSpec((DT_CB * 16, 128), lambda i: (i, 0)),
        out_shape=jax.ShapeDtypeStruct((130000, 128), jnp.float32),
        compiler_params=pltpu.CompilerParams(fuse_transposed_lhs_in_matmul=True),
    )(emT)


def _sc_gather(emb, idx_flat, tot):
    per_w = tot // NW
    nch = per_w // CHUNK
    assert nch == 2
    mesh = plsc.VectorSubcoreMesh(core_axis_name="c", subcore_axis_name="s")

    @functools.partial(
        pl.kernel,
        mesh=mesh,
        out_type=jax.ShapeDtypeStruct((tot, D), jnp.float32),
        compiler_params=pltpu.CompilerParams(use_tc_tiling_on_sc=False),
        scratch_types=[
            pltpu.VMEM((per_w,), jnp.int32),
            pltpu.VMEM((CHUNK, D), jnp.float32),
            pltpu.VMEM((CHUNK, D), jnp.float32),
            pltpu.SemaphoreType.DMA,
            pltpu.SemaphoreType.DMA,
        ],
    )
    def k(emb_hbm, idx_hbm, out_hbm, idx_v, rows_a, rows_b, sem_a, sem_b):
        wid = lax.axis_index("s") * NC + lax.axis_index("c")
        base = wid * per_w
        pltpu.sync_copy(idx_hbm.at[pl.ds(base, per_w)], idx_v)

        bufs = (rows_a, rows_b)
        sems = (sem_a, sem_b)
        cps = []
        for ci in range(2):
            cps.append([
                pltpu.async_copy(
                    emb_hbm.at[idx_v.at[pl.ds(ci * CHUNK + j * SUB, SUB)]],
                    bufs[ci].at[pl.ds(j * SUB, SUB)],
                    sems[ci],
                )
                for j in range(NSUB)
            ])
        for ci in range(2):
            for cp in cps[ci]:
                cp.wait()
            pltpu.sync_copy(bufs[ci], out_hbm.at[pl.ds(base + ci * CHUNK, CHUNK)])

    return k(emb, idx_flat)


BT = 256
GRID = B // BT


def _tc_mlp(g2, w1t, b1c, w2t, b2c, w3, b3c):
    grid = g2.shape[0] // BT

    def body(g_ref, w1_ref, b1_ref, w2_ref, b2_ref, w3_ref, b3_ref, out_ref):
        xv = g_ref[...]
        xvT = xv.T
        parts = [xvT]
        for i in range(F - 1):
            a = xvT[i * D:(i + 1) * D, :]
            rest = xvT[(i + 1) * D:, :]
            m = rest.reshape(F - 1 - i, D, BT) * a[None, :, :]
            parts.append(m.sum(axis=1))
        hT = jnp.concatenate(parts, axis=0)
        h1 = jnp.maximum(
            jnp.dot(w1_ref[...], hT, preferred_element_type=jnp.float32)
            + b1_ref[...], 0.0)
        h2 = jnp.maximum(
            jnp.dot(w2_ref[...], h1, preferred_element_type=jnp.float32)
            + b2_ref[...], 0.0)
        s = jnp.sum(h2 * w3_ref[...], axis=0)
        out_ref[...] = s.reshape(1, 1, BT) + b3_ref[...]

    return pl.pallas_call(
        body,
        grid=(grid,),
        in_specs=[
            pl.BlockSpec((BT, EMB), lambda i: (i, 0)),
            pl.BlockSpec((H1, DNN_IN), lambda i: (0, 0)),
            pl.BlockSpec((H1, 1), lambda i: (0, 0)),
            pl.BlockSpec((H2, H1), lambda i: (0, 0)),
            pl.BlockSpec((H2, 1), lambda i: (0, 0)),
            pl.BlockSpec((H2, 1), lambda i: (0, 0)),
            pl.BlockSpec((1, 1), lambda i: (0, 0)),
        ],
        out_specs=pl.BlockSpec((1, 1, BT), lambda i: (i, 0, 0)),
        out_shape=jax.ShapeDtypeStruct((grid, 1, BT), jnp.float32),
    )(g2, w1t, b1c, w2t, b2c, w3, b3c)


def kernel(x, embedding, arch, W1, b1, W2, b2, W3, b3):
    arch_f = arch.astype(jnp.float32)
    rows_i, cols_i = np.triu_indices(F, 1)
    flat_scale = jnp.repeat(arch_f, D)
    pair_scale = arch_f[rows_i] * arch_f[cols_i]
    col_scale = jnp.concatenate([flat_scale, pair_scale])
    w1t = (W1 * col_scale[:, None]).T

    table_lin = _detile(embedding.T).reshape(1040000, D)
    q = x % (DT_CB * 128)
    m2 = (x // (DT_CB * 128)) * (DT_CB * 128) + (q % (DT_CB * 16)) * 8 + q // (DT_CB * 16)
    m = m2.reshape(-1)

    parts = []
    b1c, b2c, b3c = b1.reshape(H1, 1), b2.reshape(H2, 1), b3.reshape(1, 1)
    w2t = W2.T
    NP = 4
    for h in range(NP):
        mh = lax.slice(m, (h * (TOT // NP),), ((h + 1) * (TOT // NP),))
        gh = _sc_gather(table_lin, mh, TOT // NP)
        oh = _tc_mlp(gh.reshape(B // NP, EMB), w1t, b1c, w2t, b2c, W3, b3c)
        parts.append(oh.reshape(B // NP))
    return jnp.concatenate(parts)


# speedup vs baseline: 1.0284x; 1.0284x over previous
import functools

import numpy as np
import jax
import jax.numpy as jnp
from jax import lax
from jax.experimental import pallas as pl
from jax.experimental.pallas import tpu as pltpu
from jax.experimental.pallas import tpu_sc as plsc

F = 26
D = 16
B = 16384
H1 = 256
H2 = 256
EMB = F * D
NPAIR = F * (F - 1) // 2
DNN_IN = EMB + NPAIR

NC = 2
NS = 16
NW = NC * NS
TOT = B * F
SUB = 128
CHUNK = 13 * SUB
NSUB = CHUNK // SUB


DT_CB = 125
DT_GRID = 8125 // DT_CB


def _detile_body(g_ref, out_ref):
    x = g_ref[...]
    S = DT_CB * 16
    x8 = jnp.concatenate([x[:, p * S:(p + 1) * S] for p in range(8)], axis=0)
    out_ref[...] = x8.T


def _detile(emT):
    return pl.pallas_call(
        _detile_body,
        grid=(DT_GRID,),
        in_specs=[pl.BlockSpec((16, DT_CB * 128), lambda i: (0, i))],
        out_specs=pl.BlockSpec((DT_CB * 16, 128), lambda i: (i, 0)),
        out_shape=jax.ShapeDtypeStruct((130000, 128), jnp.float32),
        compiler_params=pltpu.CompilerParams(fuse_transposed_lhs_in_matmul=True),
    )(emT)


def _sc_gather(emb, idx_flat, tot):
    per_w = tot // NW
    nch = per_w // CHUNK
    mesh = plsc.VectorSubcoreMesh(core_axis_name="c", subcore_axis_name="s")

    @functools.partial(
        pl.kernel,
        mesh=mesh,
        out_type=jax.ShapeDtypeStruct((tot, D), jnp.float32),
        compiler_params=pltpu.CompilerParams(use_tc_tiling_on_sc=False),
        scratch_types=[
            pltpu.VMEM((per_w,), jnp.int32),
            pltpu.VMEM((CHUNK, D), jnp.float32),
            pltpu.VMEM((CHUNK, D), jnp.float32),
            pltpu.SemaphoreType.DMA,
            pltpu.SemaphoreType.DMA,
        ],
    )
    def k(emb_hbm, idx_hbm, out_hbm, idx_v, rows_a, rows_b, sem_a, sem_b):
        wid = lax.axis_index("s") * NC + lax.axis_index("c")
        base = wid * per_w
        pltpu.sync_copy(idx_hbm.at[pl.ds(base, per_w)], idx_v)

        bufs = (rows_a, rows_b)
        sems = (sem_a, sem_b)

        def fire(ci):
            sl = ci & 1
            return [
                pltpu.async_copy(
                    emb_hbm.at[idx_v.at[pl.ds(ci * CHUNK + j * SUB, SUB)]],
                    bufs[sl].at[pl.ds(j * SUB, SUB)],
                    sems[sl],
                )
                for j in range(NSUB)
            ]

        def drain(ci, cps):
            for cp in cps:
                cp.wait()
            pltpu.sync_copy(bufs[ci & 1],
                            out_hbm.at[pl.ds(base + ci * CHUNK, CHUNK)])

        pend = fire(0)
        for ci in range(1, nch):
            nxt = fire(ci)
            drain(ci - 1, pend)
            pend = nxt
        drain(nch - 1, pend)

    return k(emb, idx_flat)


BT = 256
GRID = B // BT


def _tc_mlp(g2, w1t, b1c, w2t, b2c, w3, b3c):
    grid = g2.shape[0] // BT

    def body(g_ref, w1_ref, b1_ref, w2_ref, b2_ref, w3_ref, b3_ref, out_ref):
        xv = g_ref[...]
        xvT = xv.T
        parts = [xvT]
        for i in range(F - 1):
            a = xvT[i * D:(i + 1) * D, :]
            rest = xvT[(i + 1) * D:, :]
            m = rest.reshape(F - 1 - i, D, BT) * a[None, :, :]
            parts.append(m.sum(axis=1))
        hT = jnp.concatenate(parts, axis=0)
        h1 = jnp.maximum(
            jnp.dot(w1_ref[...], hT, preferred_element_type=jnp.float32)
            + b1_ref[...], 0.0)
        h2 = jnp.maximum(
            jnp.dot(w2_ref[...], h1, preferred_element_type=jnp.float32)
            + b2_ref[...], 0.0)
        s = jnp.sum(h2 * w3_ref[...], axis=0)
        out_ref[...] = s.reshape(1, 1, BT) + b3_ref[...]

    return pl.pallas_call(
        body,
        grid=(grid,),
        in_specs=[
            pl.BlockSpec((BT, EMB), lambda i: (i, 0)),
            pl.BlockSpec((H1, DNN_IN), lambda i: (0, 0)),
            pl.BlockSpec((H1, 1), lambda i: (0, 0)),
            pl.BlockSpec((H2, H1), lambda i: (0, 0)),
            pl.BlockSpec((H2, 1), lambda i: (0, 0)),
            pl.BlockSpec((H2, 1), lambda i: (0, 0)),
            pl.BlockSpec((1, 1), lambda i: (0, 0)),
        ],
        out_specs=pl.BlockSpec((1, 1, BT), lambda i: (i, 0, 0)),
        out_shape=jax.ShapeDtypeStruct((grid, 1, BT), jnp.float32),
    )(g2, w1t, b1c, w2t, b2c, w3, b3c)


def kernel(x, embedding, arch, W1, b1, W2, b2, W3, b3):
    arch_f = arch.astype(jnp.float32)
    rows_i, cols_i = np.triu_indices(F, 1)
    flat_scale = jnp.repeat(arch_f, D)
    pair_scale = arch_f[rows_i] * arch_f[cols_i]
    col_scale = jnp.concatenate([flat_scale, pair_scale])
    w1t = (W1 * col_scale[:, None]).T

    table_lin = _detile(embedding.T).reshape(1040000, D)
    q = x % (DT_CB * 128)
    m2 = (x // (DT_CB * 128)) * (DT_CB * 128) + (q % (DT_CB * 16)) * 8 + q // (DT_CB * 16)
    m = m2.reshape(-1)

    halves = []
    b1c, b2c, b3c = b1.reshape(H1, 1), b2.reshape(H2, 1), b3.reshape(1, 1)
    w2t = W2.T
    for h in range(2):
        mh = lax.slice(m, (h * (TOT // 2),), ((h + 1) * (TOT // 2),))
        gh = _sc_gather(table_lin, mh, TOT // 2)
        oh = _tc_mlp(gh.reshape(B // 2, EMB), w1t, b1c, w2t, b2c, W3, b3c)
        halves.append(oh.reshape(B // 2))
    return jnp.concatenate(halves)


# speedup vs baseline: 1.0710x; 1.0414x over previous
import functools

import numpy as np
import jax
import jax.numpy as jnp
from jax import lax
from jax.experimental import pallas as pl
from jax.experimental.pallas import tpu as pltpu
from jax.experimental.pallas import tpu_sc as plsc

F = 26
D = 16
B = 16384
H1 = 256
H2 = 256
EMB = F * D
NPAIR = F * (F - 1) // 2
DNN_IN = EMB + NPAIR

NC = 2
NS = 16
NW = NC * NS
TOT = B * F
SUB = 128
CHUNK = 13 * SUB
NSUB = CHUNK // SUB


DT_CB = 125
DT_GRID = 8125 // DT_CB


def _detile_body(g_ref, out_ref):
    x = g_ref[...]
    S = DT_CB * 16
    x8 = jnp.concatenate([x[:, p * S:(p + 1) * S] for p in range(8)], axis=0)
    out_ref[...] = x8.T


def _detile(emT):
    return pl.pallas_call(
        _detile_body,
        grid=(DT_GRID,),
        in_specs=[pl.BlockSpec((16, DT_CB * 128), lambda i: (0, i))],
        out_specs=pl.BlockSpec((DT_CB * 16, 128), lambda i: (i, 0)),
        out_shape=jax.ShapeDtypeStruct((130000, 128), jnp.float32),
        compiler_params=pltpu.CompilerParams(fuse_transposed_lhs_in_matmul=True),
    )(emT)


def _sc_gather(emb, idx_flat, tot):
    per_w = tot // NW
    nch = per_w // CHUNK
    mesh = plsc.VectorSubcoreMesh(core_axis_name="c", subcore_axis_name="s")

    @functools.partial(
        pl.kernel,
        mesh=mesh,
        out_type=jax.ShapeDtypeStruct((tot, D), jnp.float32),
        compiler_params=pltpu.CompilerParams(use_tc_tiling_on_sc=False),
        scratch_types=[
            pltpu.VMEM((per_w,), jnp.int32),
            pltpu.VMEM((CHUNK, D), jnp.float32),
            pltpu.VMEM((CHUNK, D), jnp.float32),
            pltpu.SemaphoreType.DMA,
            pltpu.SemaphoreType.DMA,
        ],
    )
    def k(emb_hbm, idx_hbm, out_hbm, idx_v, rows_a, rows_b, sem_a, sem_b):
        wid = lax.axis_index("s") * NC + lax.axis_index("c")
        base = wid * per_w
        pltpu.sync_copy(idx_hbm.at[pl.ds(base, per_w)], idx_v)

        bufs = (rows_a, rows_b)
        sems = (sem_a, sem_b)

        def fire(ci):
            sl = ci & 1
            return [
                pltpu.async_copy(
                    emb_hbm.at[idx_v.at[pl.ds(ci * CHUNK + j * SUB, SUB)]],
                    bufs[sl].at[pl.ds(j * SUB, SUB)],
                    sems[sl],
                )
                for j in range(NSUB)
            ]

        def drain(ci, cps):
            for cp in cps:
                cp.wait()
            pltpu.sync_copy(bufs[ci & 1],
                            out_hbm.at[pl.ds(base + ci * CHUNK, CHUNK)])

        pend = fire(0)
        for ci in range(1, nch):
            nxt = fire(ci)
            drain(ci - 1, pend)
            pend = nxt
        drain(nch - 1, pend)

    return k(emb, idx_flat)


BT = 512
GRID = B // BT


def _tc_mlp(g2, w1t, b1c, w2t, b2c, w3, b3c):
    grid = g2.shape[0] // BT

    def body(g_ref, w1_ref, b1_ref, w2_ref, b2_ref, w3_ref, b3_ref, out_ref):
        xv = g_ref[...]
        xvT = xv.T
        parts = [xvT]
        for i in range(F - 1):
            a = xvT[i * D:(i + 1) * D, :]
            rest = xvT[(i + 1) * D:, :]
            m = rest.reshape(F - 1 - i, D, BT) * a[None, :, :]
            parts.append(m.sum(axis=1))
        hT = jnp.concatenate(parts, axis=0)
        h1 = jnp.maximum(
            jnp.dot(w1_ref[...], hT, preferred_element_type=jnp.float32)
            + b1_ref[...], 0.0)
        h2 = jnp.maximum(
            jnp.dot(w2_ref[...], h1, preferred_element_type=jnp.float32)
            + b2_ref[...], 0.0)
        s = jnp.sum(h2 * w3_ref[...], axis=0)
        out_ref[...] = s.reshape(1, 1, BT) + b3_ref[...]

    return pl.pallas_call(
        body,
        grid=(grid,),
        in_specs=[
            pl.BlockSpec((BT, EMB), lambda i: (i, 0)),
            pl.BlockSpec((H1, DNN_IN), lambda i: (0, 0)),
            pl.BlockSpec((H1, 1), lambda i: (0, 0)),
            pl.BlockSpec((H2, H1), lambda i: (0, 0)),
            pl.BlockSpec((H2, 1), lambda i: (0, 0)),
            pl.BlockSpec((H2, 1), lambda i: (0, 0)),
            pl.BlockSpec((1, 1), lambda i: (0, 0)),
        ],
        out_specs=pl.BlockSpec((1, 1, BT), lambda i: (i, 0, 0)),
        out_shape=jax.ShapeDtypeStruct((grid, 1, BT), jnp.float32),
    )(g2, w1t, b1c, w2t, b2c, w3, b3c)


def kernel(x, embedding, arch, W1, b1, W2, b2, W3, b3):
    arch_f = arch.astype(jnp.float32)
    rows_i, cols_i = np.triu_indices(F, 1)
    flat_scale = jnp.repeat(arch_f, D)
    pair_scale = arch_f[rows_i] * arch_f[cols_i]
    col_scale = jnp.concatenate([flat_scale, pair_scale])
    w1t = (W1 * col_scale[:, None]).T

    table_lin = _detile(embedding.T).reshape(1040000, D)
    q = x % (DT_CB * 128)
    m2 = (x // (DT_CB * 128)) * (DT_CB * 128) + (q % (DT_CB * 16)) * 8 + q // (DT_CB * 16)
    m = m2.reshape(-1)

    halves = []
    b1c, b2c, b3c = b1.reshape(H1, 1), b2.reshape(H2, 1), b3.reshape(1, 1)
    w2t = W2.T
    for h in range(2):
        mh = lax.slice(m, (h * (TOT // 2),), ((h + 1) * (TOT // 2),))
        gh = _sc_gather(table_lin, mh, TOT // 2)
        oh = _tc_mlp(gh.reshape(B // 2, EMB), w1t, b1c, w2t, b2c, W3, b3c)
        halves.append(oh.reshape(B // 2))
    return jnp.concatenate(halves)
